# baseline (speedup 1.0000x reference)
import functools

import jax
import jax.numpy as jnp
from jax import lax
from jax.experimental import pallas as pl
from jax.experimental.pallas import tpu as pltpu
from jax.experimental.pallas import tpu_sc as plsc

N = 16384
S = 200
E = 32
NC = 2
NS = 16
NW = NC * NS
NPW = N // NW
NBW = NPW // 128
ROW_HI = E // 8

_mesh = plsc.VectorSubcoreMesh(core_axis_name="c", subcore_axis_name="s")


@functools.partial(
    pl.kernel,
    mesh=_mesh,
    out_type=jax.ShapeDtypeStruct((S, ROW_HI, (N // 128) * 8 * 128),
                                  jnp.float32),
    compiler_params=pltpu.CompilerParams(use_tc_tiling_on_sc=False,
                                         needs_layout_passes=False),
    scratch_types=[
        pltpu.VMEM((NPW,), jnp.int32),
        pltpu.VMEM((NPW,), jnp.int32),
        pltpu.VMEM((NPW,), jnp.int32),
        pltpu.VMEM((NPW,), jnp.int32),
        pltpu.VMEM((NPW, E), jnp.float32),
        pltpu.VMEM((NPW, E), jnp.float32),
        pltpu.VMEM((ROW_HI, NBW * 8 * 128), jnp.float32),
        pltpu.VMEM((ROW_HI, NBW * 8 * 128), jnp.float32),
        pltpu.SemaphoreType.DMA,
        pltpu.SemaphoreType.DMA,
        pltpu.SemaphoreType.DMA,
        pltpu.SemaphoreType.DMA,
        pltpu.SemaphoreType.DMA,
        pltpu.SemaphoreType.DMA,
        pltpu.SemaphoreType.DMA,
        pltpu.SemaphoreType.DMA,
    ],
)
def _gather_t(idx_hbm, table_hbm, out_hbm,
              ib0, ib1, ib2, ib3, rows0, rows1, tb0, tb1,
              is0, is1, is2, is3, gs0, gs1, os0, os1):
    wid = lax.axis_index("s") * NC + lax.axis_index("c")
    n0 = wid * NPW

    ibuf = (ib0, ib1, ib2, ib3)
    isem = (is0, is1, is2, is3)
    rows = (rows0, rows1)
    gsem = (gs0, gs1)
    tbuf = (tb0, tb1)
    osem = (os0, os1)

    i16 = lax.broadcasted_iota(jnp.int32, (16,), 0)
    ebv0 = i16 >> 3
    ebv1 = ebv0 + 2
    vf = (i16 & 7) * 128

    def start_idx(s, k):
        pltpu.async_copy(idx_hbm.at[s, pl.ds(n0, NPW)], ibuf[k], isem[k])

    def wait_idx(s, k):
        pltpu.make_async_copy(idx_hbm.at[s, pl.ds(n0, NPW)], ibuf[k],
                              isem[k]).wait()

    def start_gather(k, b):
        pltpu.async_copy(table_hbm.at[ibuf[k]], rows[b], gsem[b])

    def wait_gather(k, b):
        pltpu.make_async_copy(table_hbm.at[ibuf[k]], rows[b],
                              gsem[b]).wait()

    def transpose(b):
        rows_b, tb = rows[b], tbuf[b]

        for nb in range(NBW):
            base = nb * 1024

            def trow(ni, c, nb=nb, base=base):
                n = nb * 128 + ni
                fl = vf + (base + ni)
                plsc.store_scatter(tb, [ebv0, fl], rows_b[n, pl.ds(0, 16)])
                plsc.store_scatter(tb, [ebv1, fl], rows_b[n, pl.ds(16, 16)])
                return c

            lax.fori_loop(0, 128, trow, 0, unroll=8)

    def start_out(s, b):
        pltpu.async_copy(tbuf[b], out_hbm.at[s, :, pl.ds(n0 * 8, NPW * 8)],
                         osem[b])

    def wait_out(s, b):
        pltpu.make_async_copy(tbuf[b],
                              out_hbm.at[s, :, pl.ds(n0 * 8, NPW * 8)],
                              osem[b]).wait()

    start_idx(0, 0)
    start_idx(1, 1)
    wait_idx(0, 0)
    start_gather(0, 0)
    start_idx(2, 2)
    wait_idx(1, 1)
    start_gather(1, 1)
    start_idx(3, 3)
    wait_gather(0, 0)
    transpose(0)
    start_out(0, 0)

    def quad(q, carry):
        s_base = 4 * q + 2
        for j, (k, b) in enumerate([(2, 0), (3, 1), (0, 0), (1, 1)]):
            s = s_base + j
            wait_idx(s, k)
            wait_out(s - 2, b)
            start_gather(k, b)
            start_idx(s + 2, (k + 2) % 4)
            wait_gather((k - 1) % 4, 1 - b)
            transpose(1 - b)
            start_out(s - 1, 1 - b)
        return carry

    lax.fori_loop(0, (S - 4) // 4, quad, 0, unroll=False)

    for (s, k, b) in [(S - 2, 2, 0), (S - 1, 3, 1)]:
        wait_idx(s, k)
        wait_out(s - 2, b)
        start_gather(k, b)
        wait_gather((k - 1) % 4, 1 - b)
        transpose(1 - b)
        start_out(s - 1, 1 - b)
    wait_gather(3, 1)
    transpose(1)
    start_out(S - 1, 1)
    wait_out(S - 2, 0)
    wait_out(S - 1, 1)


def kernel(atom_type_indices, embedding_table):
    idx_t = atom_type_indices.T
    o = _gather_t(idx_t, embedding_table)
    o = o.reshape(S, ROW_HI, N // 128, 8, 128)
    return o.transpose(2, 4, 0, 1, 3).reshape(N, S, E)


# speedup vs baseline: 1.3613x; 1.3613x over previous
import functools

import jax
import jax.numpy as jnp
from jax import lax
from jax.experimental import pallas as pl
from jax.experimental.pallas import tpu as pltpu
from jax.experimental.pallas import tpu_sc as plsc

N = 16384
S = 200
E = 32
NC = 2
NS = 16
NW = NC * NS
NPW = N // NW
NBW = NPW // 128
ROW_HI = E // 8

_mesh = plsc.VectorSubcoreMesh(core_axis_name="c", subcore_axis_name="s")


@functools.partial(
    pl.kernel,
    mesh=_mesh,
    out_type=jax.ShapeDtypeStruct((S, ROW_HI, (N // 128) * 8 * 128),
                                  jnp.float32),
    compiler_params=pltpu.CompilerParams(use_tc_tiling_on_sc=False,
                                         needs_layout_passes=False),
    scratch_types=[
        pltpu.VMEM((NPW,), jnp.int32),
        pltpu.VMEM((NPW,), jnp.int32),
        pltpu.VMEM((NPW,), jnp.int32),
        pltpu.VMEM((NPW,), jnp.int32),
        pltpu.VMEM((NPW, E), jnp.float32),
        pltpu.VMEM((NPW, E), jnp.float32),
        pltpu.VMEM((ROW_HI, NBW * 8 * 128), jnp.float32),
        pltpu.VMEM((ROW_HI, NBW * 8 * 128), jnp.float32),
        pltpu.SemaphoreType.DMA,
        pltpu.SemaphoreType.DMA,
        pltpu.SemaphoreType.DMA,
        pltpu.SemaphoreType.DMA,
        pltpu.SemaphoreType.DMA,
        pltpu.SemaphoreType.DMA,
        pltpu.SemaphoreType.DMA,
        pltpu.SemaphoreType.DMA,
    ],
)
def _gather_t(idx_hbm, table_hbm, out_hbm,
              ib0, ib1, ib2, ib3, rows0, rows1, tb0, tb1,
              is0, is1, is2, is3, gs0, gs1, os0, os1):
    wid = lax.axis_index("s") * NC + lax.axis_index("c")
    n0 = wid * NPW

    ibuf = (ib0, ib1, ib2, ib3)
    isem = (is0, is1, is2, is3)
    rows = (rows0, rows1)
    gsem = (gs0, gs1)
    tbuf = (tb0, tb1)
    osem = (os0, os1)

    i16 = lax.broadcasted_iota(jnp.int32, (16,), 0)
    ebv0 = i16 >> 3
    ebv1 = ebv0 + 2
    vf = (i16 & 7) * 128

    def start_idx(s, k):
        pltpu.async_copy(idx_hbm.at[s, pl.ds(n0, NPW)], ibuf[k], isem[k])

    def wait_idx(s, k):
        pltpu.make_async_copy(idx_hbm.at[s, pl.ds(n0, NPW)], ibuf[k],
                              isem[k]).wait()

    def start_gather(k, b):
        pltpu.async_copy(table_hbm.at[ibuf[k]], rows[b], gsem[b])

    def wait_gather(k, b):
        pltpu.make_async_copy(table_hbm.at[ibuf[k]], rows[b],
                              gsem[b]).wait()

    def transpose(b):
        rows_b, tb = rows[b], tbuf[b]

        for nb in range(NBW):
            base = nb * 1024

            @plsc.parallel_loop(0, 128, unroll=8)
            def trow(ni, nb=nb, base=base):
                n = nb * 128 + ni
                fl = vf + (base + ni)
                plsc.store_scatter(tb, [ebv0, fl], rows_b[n, pl.ds(0, 16)])
                plsc.store_scatter(tb, [ebv1, fl], rows_b[n, pl.ds(16, 16)])

    def start_out(s, b):
        pltpu.async_copy(tbuf[b], out_hbm.at[s, :, pl.ds(n0 * 8, NPW * 8)],
                         osem[b])

    def wait_out(s, b):
        pltpu.make_async_copy(tbuf[b],
                              out_hbm.at[s, :, pl.ds(n0 * 8, NPW * 8)],
                              osem[b]).wait()

    start_idx(0, 0)
    start_idx(1, 1)
    wait_idx(0, 0)
    start_gather(0, 0)
    start_idx(2, 2)
    wait_idx(1, 1)
    start_gather(1, 1)
    start_idx(3, 3)
    wait_gather(0, 0)
    transpose(0)
    start_out(0, 0)

    def quad(q, carry):
        s_base = 4 * q + 2
        for j, (k, b) in enumerate([(2, 0), (3, 1), (0, 0), (1, 1)]):
            s = s_base + j
            wait_idx(s, k)
            wait_out(s - 2, b)
            start_gather(k, b)
            start_idx(s + 2, (k + 2) % 4)
            wait_gather((k - 1) % 4, 1 - b)
            transpose(1 - b)
            start_out(s - 1, 1 - b)
        return carry

    lax.fori_loop(0, (S - 4) // 4, quad, 0, unroll=False)

    for (s, k, b) in [(S - 2, 2, 0), (S - 1, 3, 1)]:
        wait_idx(s, k)
        wait_out(s - 2, b)
        start_gather(k, b)
        wait_gather((k - 1) % 4, 1 - b)
        transpose(1 - b)
        start_out(s - 1, 1 - b)
    wait_gather(3, 1)
    transpose(1)
    start_out(S - 1, 1)
    wait_out(S - 2, 0)
    wait_out(S - 1, 1)


def kernel(atom_type_indices, embedding_table):
    idx_t = atom_type_indices.T
    o = _gather_t(idx_t, embedding_table)
    o = o.reshape(S, ROW_HI, N // 128, 8, 128)
    return o.transpose(2, 4, 0, 1, 3).reshape(N, S, E)


# speedup vs baseline: 5.3673x; 3.9429x over previous
import functools

import jax
import jax.numpy as jnp
from jax import lax
from jax.experimental import pallas as pl
from jax.experimental.pallas import tpu as pltpu
from jax.experimental.pallas import tpu_sc as plsc

N = 16384
S = 200
E = 32
NC = 2
NS = 16
NW = NC * NS
NPW = N // NW
NBW = NPW // 128
ROW_HI = E // 8
PITCH = 137

_mesh = plsc.VectorSubcoreMesh(core_axis_name="c", subcore_axis_name="s")


@functools.partial(
    pl.kernel,
    mesh=_mesh,
    out_type=jax.ShapeDtypeStruct((S, ROW_HI, N // 128, 8, 128),
                                  jnp.float32),
    compiler_params=pltpu.CompilerParams(use_tc_tiling_on_sc=False,
                                         needs_layout_passes=False),
    scratch_types=[
        pltpu.VMEM((NPW,), jnp.int32),
        pltpu.VMEM((NPW,), jnp.int32),
        pltpu.VMEM((NPW,), jnp.int32),
        pltpu.VMEM((NPW,), jnp.int32),
        pltpu.VMEM((NPW, E), jnp.float32),
        pltpu.VMEM((NPW, E), jnp.float32),
        pltpu.VMEM((ROW_HI, NBW, 8, PITCH), jnp.float32),
        pltpu.VMEM((ROW_HI, NBW, 8, PITCH), jnp.float32),
        pltpu.SemaphoreType.DMA,
        pltpu.SemaphoreType.DMA,
        pltpu.SemaphoreType.DMA,
        pltpu.SemaphoreType.DMA,
        pltpu.SemaphoreType.DMA,
        pltpu.SemaphoreType.DMA,
        pltpu.SemaphoreType.DMA,
        pltpu.SemaphoreType.DMA,
    ],
)
def _gather_t(idx_hbm, table_hbm, out_hbm,
              ib0, ib1, ib2, ib3, rows0, rows1, tb0, tb1,
              is0, is1, is2, is3, gs0, gs1, os0, os1):
    wid = lax.axis_index("s") * NC + lax.axis_index("c")
    n0 = wid * NPW

    ibuf = (ib0, ib1, ib2, ib3)
    isem = (is0, is1, is2, is3)
    rows = (rows0, rows1)
    gsem = (gs0, gs1)
    tbuf = (tb0, tb1)
    osem = (os0, os1)

    i16 = lax.broadcasted_iota(jnp.int32, (16,), 0)
    ebv0 = i16 >> 3
    ebv1 = ebv0 + 2
    elv = i16 & 7

    def start_idx(s, k):
        pltpu.async_copy(idx_hbm.at[s, pl.ds(n0, NPW)], ibuf[k], isem[k])

    def wait_idx(s, k):
        pltpu.make_async_copy(idx_hbm.at[s, pl.ds(n0, NPW)], ibuf[k],
                              isem[k]).wait()

    def start_gather(k, b):
        pltpu.async_copy(table_hbm.at[ibuf[k]], rows[b], gsem[b])

    def wait_gather(k, b):
        pltpu.make_async_copy(table_hbm.at[ibuf[k]], rows[b],
                              gsem[b]).wait()

    def transpose(b):
        rows_b, tb = rows[b], tbuf[b]

        for nb in range(NBW):
            nbv = jnp.full((16,), nb, jnp.int32)

            @plsc.parallel_loop(0, 128, unroll=8)
            def trow(ni, nb=nb, nbv=nbv):
                n = nb * 128 + ni
                niv = jnp.full((16,), 0, jnp.int32) + ni
                plsc.store_scatter(tb, [ebv0, nbv, elv, niv],
                                   rows_b[n, pl.ds(0, 16)])
                plsc.store_scatter(tb, [ebv1, nbv, elv, niv],
                                   rows_b[n, pl.ds(16, 16)])

    nb0 = wid * NBW

    def start_out(s, b):
        pltpu.async_copy(tbuf[b].at[:, :, :, pl.ds(0, 128)],
                         out_hbm.at[s, :, pl.ds(nb0, NBW)], osem[b])

    def wait_out(s, b):
        pltpu.make_async_copy(tbuf[b].at[:, :, :, pl.ds(0, 128)],
                              out_hbm.at[s, :, pl.ds(nb0, NBW)],
                              osem[b]).wait()

    start_idx(0, 0)
    start_idx(1, 1)
    wait_idx(0, 0)
    start_gather(0, 0)
    start_idx(2, 2)
    wait_idx(1, 1)
    start_gather(1, 1)
    start_idx(3, 3)
    wait_gather(0, 0)
    transpose(0)
    start_out(0, 0)

    def quad(q, carry):
        s_base = 4 * q + 2
        for j, (k, b) in enumerate([(2, 0), (3, 1), (0, 0), (1, 1)]):
            s = s_base + j
            wait_idx(s, k)
            wait_out(s - 2, b)
            start_gather(k, b)
            start_idx(s + 2, (k + 2) % 4)
            wait_gather((k - 1) % 4, 1 - b)
            transpose(1 - b)
            start_out(s - 1, 1 - b)
        return carry

    lax.fori_loop(0, (S - 4) // 4, quad, 0, unroll=False)

    for (s, k, b) in [(S - 2, 2, 0), (S - 1, 3, 1)]:
        wait_idx(s, k)
        wait_out(s - 2, b)
        start_gather(k, b)
        wait_gather((k - 1) % 4, 1 - b)
        transpose(1 - b)
        start_out(s - 1, 1 - b)
    wait_gather(3, 1)
    transpose(1)
    start_out(S - 1, 1)
    wait_out(S - 2, 0)
    wait_out(S - 1, 1)


def kernel(atom_type_indices, embedding_table):
    idx_t = atom_type_indices.T
    o = _gather_t(idx_t, embedding_table)
    return o.transpose(2, 4, 0, 1, 3).reshape(N, S, E)


# speedup vs baseline: 5.4162x; 1.0091x over previous
import functools

import jax
import jax.numpy as jnp
from jax import lax
from jax.experimental import pallas as pl
from jax.experimental.pallas import tpu as pltpu
from jax.experimental.pallas import tpu_sc as plsc

N = 16384
S = 200
E = 32
NC = 2
NS = 16
NW = NC * NS
NPW = N // NW
NBW = NPW // 128
ROW_HI = E // 8
PITCH = 137

_mesh = plsc.VectorSubcoreMesh(core_axis_name="c", subcore_axis_name="s")


@functools.partial(
    pl.kernel,
    mesh=_mesh,
    out_type=jax.ShapeDtypeStruct((S, ROW_HI, N // 128, 8, 128),
                                  jnp.float32),
    compiler_params=pltpu.CompilerParams(use_tc_tiling_on_sc=False,
                                         needs_layout_passes=False),
    scratch_types=[
        pltpu.VMEM((NBW, 128), jnp.int32),
        pltpu.VMEM((NBW, 128), jnp.int32),
        pltpu.VMEM((NBW, 128), jnp.int32),
        pltpu.VMEM((NBW, 128), jnp.int32),
        pltpu.VMEM((NPW, E), jnp.float32),
        pltpu.VMEM((NPW, E), jnp.float32),
        pltpu.VMEM((ROW_HI, NBW, 8, PITCH), jnp.float32),
        pltpu.VMEM((ROW_HI, NBW, 8, PITCH), jnp.float32),
        pltpu.SemaphoreType.DMA,
        pltpu.SemaphoreType.DMA,
        pltpu.SemaphoreType.DMA,
        pltpu.SemaphoreType.DMA,
        pltpu.SemaphoreType.DMA,
        pltpu.SemaphoreType.DMA,
        pltpu.SemaphoreType.DMA,
        pltpu.SemaphoreType.DMA,
    ],
)
def _gather_t(idx_hbm, table_hbm, out_hbm,
              ib0, ib1, ib2, ib3, rows0, rows1, tb0, tb1,
              is0, is1, is2, is3, gs0, gs1, os0, os1):
    wid = lax.axis_index("s") * NC + lax.axis_index("c")
    n0 = wid * NPW

    ibuf = (ib0, ib1, ib2, ib3)
    isem = (is0, is1, is2, is3)
    rows = (rows0, rows1)
    gsem = (gs0, gs1)
    tbuf = (tb0, tb1)
    osem = (os0, os1)

    i16 = lax.broadcasted_iota(jnp.int32, (16,), 0)
    ebv0 = i16 >> 3
    ebv1 = ebv0 + 2
    elv = i16 & 7

    nb0 = wid * NBW

    def start_idx(s, k):
        pltpu.async_copy(idx_hbm.at[s // 8, pl.ds(nb0, NBW), s % 8],
                         ibuf[k], isem[k])

    def wait_idx(s, k):
        pltpu.make_async_copy(idx_hbm.at[s // 8, pl.ds(nb0, NBW), s % 8],
                              ibuf[k], isem[k]).wait()

    def start_gather(k, b):
        for nb in range(NBW):
            pltpu.async_copy(table_hbm.at[ibuf[k].at[nb]],
                             rows[b].at[pl.ds(nb * 128, 128)], gsem[b])

    def wait_gather(k, b):
        for nb in range(NBW):
            pltpu.make_async_copy(table_hbm.at[ibuf[k].at[nb]],
                                  rows[b].at[pl.ds(nb * 128, 128)],
                                  gsem[b]).wait()


    def transpose(b):
        rows_b, tb = rows[b], tbuf[b]

        for nb in range(NBW):
            nbv = jnp.full((16,), nb, jnp.int32)

            @plsc.parallel_loop(0, 128, unroll=8)
            def trow(ni, nb=nb, nbv=nbv):
                n = nb * 128 + ni
                niv = jnp.full((16,), 0, jnp.int32) + ni
                plsc.store_scatter(tb, [ebv0, nbv, elv, niv],
                                   rows_b[n, pl.ds(0, 16)])
                plsc.store_scatter(tb, [ebv1, nbv, elv, niv],
                                   rows_b[n, pl.ds(16, 16)])

    def start_out(s, b):
        pltpu.async_copy(tbuf[b].at[:, :, :, pl.ds(0, 128)],
                         out_hbm.at[s, :, pl.ds(nb0, NBW)], osem[b])

    def wait_out(s, b):
        pltpu.make_async_copy(tbuf[b].at[:, :, :, pl.ds(0, 128)],
                              out_hbm.at[s, :, pl.ds(nb0, NBW)],
                              osem[b]).wait()

    start_idx(0, 0)
    start_idx(1, 1)
    wait_idx(0, 0)
    start_gather(0, 0)
    start_idx(2, 2)
    wait_idx(1, 1)
    start_gather(1, 1)
    start_idx(3, 3)
    wait_gather(0, 0)
    transpose(0)
    start_out(0, 0)

    def quad(q, carry):
        s_base = 4 * q + 2
        for j, (k, b) in enumerate([(2, 0), (3, 1), (0, 0), (1, 1)]):
            s = s_base + j
            wait_idx(s, k)
            wait_out(s - 2, b)
            start_gather(k, b)
            start_idx(s + 2, (k + 2) % 4)
            wait_gather((k - 1) % 4, 1 - b)
            transpose(1 - b)
            start_out(s - 1, 1 - b)
        return carry

    lax.fori_loop(0, (S - 4) // 4, quad, 0, unroll=False)

    for (s, k, b) in [(S - 2, 2, 0), (S - 1, 3, 1)]:
        wait_idx(s, k)
        wait_out(s - 2, b)
        start_gather(k, b)
        wait_gather((k - 1) % 4, 1 - b)
        transpose(1 - b)
        start_out(s - 1, 1 - b)
    wait_gather(3, 1)
    transpose(1)
    start_out(S - 1, 1)
    wait_out(S - 2, 0)
    wait_out(S - 1, 1)


def kernel(atom_type_indices, embedding_table):
    idx4 = (atom_type_indices.reshape(128, 128, 25, 8)
            .transpose(2, 0, 3, 1))
    o = _gather_t(idx4, embedding_table)
    return o.transpose(2, 4, 0, 1, 3).reshape(N, S, E)


# speedup vs baseline: 5.7129x; 1.0548x over previous
import functools

import jax
import jax.numpy as jnp
from jax import lax
from jax.experimental import pallas as pl
from jax.experimental.pallas import tpu as pltpu
from jax.experimental.pallas import tpu_sc as plsc

N = 16384
S = 200
E = 32
NC = 2
NS = 16
NW = NC * NS
NPW = N // NW
NBW = NPW // 128
ROW_HI = E // 8
PITCH = 137

_mesh = plsc.VectorSubcoreMesh(core_axis_name="c", subcore_axis_name="s")


@functools.partial(
    pl.kernel,
    mesh=_mesh,
    out_type=jax.ShapeDtypeStruct((S, ROW_HI, N // 128, 8, 128),
                                  jnp.float32),
    compiler_params=pltpu.CompilerParams(use_tc_tiling_on_sc=False,
                                         needs_layout_passes=False),
    scratch_types=[
        pltpu.VMEM((NBW, 128), jnp.int32),
        pltpu.VMEM((NBW, 128), jnp.int32),
        pltpu.VMEM((NBW, 128), jnp.int32),
        pltpu.VMEM((NBW, 128), jnp.int32),
        pltpu.VMEM((NPW, E), jnp.float32),
        pltpu.VMEM((NPW, E), jnp.float32),
        pltpu.VMEM((NPW, E), jnp.float32),
        pltpu.VMEM((NPW, E), jnp.float32),
        pltpu.VMEM((ROW_HI, NBW, 8, PITCH), jnp.float32),
        pltpu.VMEM((ROW_HI, NBW, 8, PITCH), jnp.float32),
        pltpu.SemaphoreType.DMA,
        pltpu.SemaphoreType.DMA,
        pltpu.SemaphoreType.DMA,
        pltpu.SemaphoreType.DMA,
        pltpu.SemaphoreType.DMA,
        pltpu.SemaphoreType.DMA,
        pltpu.SemaphoreType.DMA,
        pltpu.SemaphoreType.DMA,
        pltpu.SemaphoreType.DMA,
        pltpu.SemaphoreType.DMA,
    ],
)
def _gather_t(idx_hbm, table_hbm, out_hbm,
              ib0, ib1, ib2, ib3, rows0, rows1, rows2, rows3, tb0, tb1,
              is0, is1, is2, is3, gs0, gs1, gs2, gs3, os0, os1):
    wid = lax.axis_index("s") * NC + lax.axis_index("c")
    nb0 = wid * NBW

    ibuf = (ib0, ib1, ib2, ib3)
    isem = (is0, is1, is2, is3)
    rows = (rows0, rows1, rows2, rows3)
    gsem = (gs0, gs1, gs2, gs3)
    tbuf = (tb0, tb1)
    osem = (os0, os1)

    i16 = lax.broadcasted_iota(jnp.int32, (16,), 0)
    ebv0 = i16 >> 3
    ebv1 = ebv0 + 2
    elv = i16 & 7

    def start_idx(s, k):
        pltpu.async_copy(idx_hbm.at[s // 8, pl.ds(nb0, NBW), s % 8],
                         ibuf[k], isem[k])

    def wait_idx(s, k):
        pltpu.make_async_copy(idx_hbm.at[s // 8, pl.ds(nb0, NBW), s % 8],
                              ibuf[k], isem[k]).wait()

    def start_gather(k, r):
        for nb in range(NBW):
            pltpu.async_copy(table_hbm.at[ibuf[k].at[nb]],
                             rows[r].at[pl.ds(nb * 128, 128)], gsem[r])

    def wait_gather(k, r):
        for nb in range(NBW):
            pltpu.make_async_copy(table_hbm.at[ibuf[k].at[nb]],
                                  rows[r].at[pl.ds(nb * 128, 128)],
                                  gsem[r]).wait()

    def transpose(r, t):
        rows_b, tb = rows[r], tbuf[t]

        for nb in range(NBW):
            nbv = jnp.full((16,), nb, jnp.int32)

            @plsc.parallel_loop(0, 128, unroll=8)
            def trow(ni, nb=nb, nbv=nbv):
                n = nb * 128 + ni
                niv = jnp.full((16,), 0, jnp.int32) + ni
                plsc.store_scatter(tb, [ebv0, nbv, elv, niv],
                                   rows_b[n, pl.ds(0, 16)])
                plsc.store_scatter(tb, [ebv1, nbv, elv, niv],
                                   rows_b[n, pl.ds(16, 16)])

    def start_out(s, t):
        pltpu.async_copy(tbuf[t].at[:, :, :, pl.ds(0, 128)],
                         out_hbm.at[s, :, pl.ds(nb0, NBW)], osem[t])

    def wait_out(s, t):
        pltpu.make_async_copy(tbuf[t].at[:, :, :, pl.ds(0, 128)],
                              out_hbm.at[s, :, pl.ds(nb0, NBW)],
                              osem[t]).wait()

    def col(s, j, has_out_wait=True, idx_ahead=True, gather_ahead=True):
        k, r, t = j, j, j % 2
        if gather_ahead:
            k2 = (j + 2) % 4
            wait_idx(s + 2, k2)
            start_gather(k2, k2)
        if idx_ahead:
            start_idx(s + 3, (j + 3) % 4)
        if has_out_wait:
            wait_out(s - 2, t)
        wait_gather(k, r)
        transpose(r, t)
        start_out(s, t)

    start_idx(0, 0)
    start_idx(1, 1)
    wait_idx(0, 0)
    start_gather(0, 0)
    start_idx(2, 2)
    wait_idx(1, 1)
    start_gather(1, 1)

    col(0, 0, has_out_wait=False)
    col(1, 1, has_out_wait=False)
    col(2, 2)
    col(3, 3)

    def quad(q, carry):
        s_base = 4 * q
        for j in range(4):
            col(s_base + j, j)
        return carry

    lax.fori_loop(1, 48, quad, 0, unroll=False)

    for s in range(192, 197):
        col(s, s % 4)
    col(197, 1, idx_ahead=False)
    col(198, 2, idx_ahead=False, gather_ahead=False)
    col(199, 3, idx_ahead=False, gather_ahead=False)
    wait_out(198, 0)
    wait_out(199, 1)


def kernel(atom_type_indices, embedding_table):
    idx4 = (atom_type_indices.reshape(128, 128, 25, 8)
            .transpose(2, 0, 3, 1))
    o = _gather_t(idx4, embedding_table)
    return o.transpose(2, 4, 0, 1, 3).reshape(N, S, E)
